# baseline (speedup 1.0000x reference)
import functools

import jax
import jax.numpy as jnp
from jax import lax
from jax.experimental import pallas as pl
from jax.experimental.pallas import tpu as pltpu
from jax.experimental.pallas import tpu_sc as plsc

VOCAB = 100000
N_CONSTS = 16384
N_ATOMS = 65536
N_PREDS = 64
D_CONST = 128
D_ATOM = 64

NC = 2
NS = 16
NW = NC * NS
LANES = 16
N_CHUNKS = 2
CA = N_ATOMS // N_CHUNKS
CHUNK = CA // NW
BLK = 128
NBLK = CHUNK // BLK


def _sc_interact(x_default, head_pos, tail_pos, constant_table):
    mesh = plsc.VectorSubcoreMesh(core_axis_name="c", subcore_axis_name="s",
                                  num_cores=2)
    hpos3 = head_pos.reshape(NW, NBLK, BLK)
    tpos3 = tail_pos.reshape(NW, NBLK, BLK)

    @functools.partial(
        pl.kernel,
        mesh=mesh,
        compiler_params=pltpu.CompilerParams(needs_layout_passes=False),
        out_type=jax.ShapeDtypeStruct((CA, D_CONST // 2), jnp.int32),
        scratch_types=[
            pltpu.VMEM((NBLK, BLK), jnp.int32),
            pltpu.VMEM((NBLK, BLK), jnp.int32),
            pltpu.VMEM((NBLK, BLK), jnp.int32),
            pltpu.VMEM((NBLK, BLK), jnp.int32),
            pltpu.VMEM((2, BLK, D_CONST), jnp.float32),
            pltpu.VMEM((2, BLK, D_CONST), jnp.float32),
            pltpu.VMEM((2, BLK, D_CONST // 2), jnp.int32),
            pltpu.SemaphoreType.DMA,
            pltpu.SemaphoreType.DMA,
            pltpu.SemaphoreType.DMA,
            pltpu.SemaphoreType.DMA,
            pltpu.SemaphoreType.DMA,
        ],
    )
    def k(xdef_hbm, hpos_hbm, tpos_hbm, table_hbm, q_hbm,
          hpos_v, tpos_v, hid_v, tid_v, h_v, t_v, q_v,
          sg0, sg1, sw0, sw1, sem_t):
        wid = lax.axis_index("s") * NC + lax.axis_index("c")
        base = pl.multiple_of(wid * CHUNK, CHUNK)
        pltpu.sync_copy(hpos_hbm.at[wid], hpos_v)
        pltpu.sync_copy(tpos_hbm.at[wid], tpos_v)

        trans = []
        for j in range(NBLK):
            trans.append(
                pltpu.async_copy(xdef_hbm.at[hpos_v.at[j]], hid_v.at[j], sem_t))
            trans.append(
                pltpu.async_copy(xdef_hbm.at[tpos_v.at[j]], tid_v.at[j], sem_t))
        for cp in trans:
            cp.wait()

        sg = (sg0, sg1)
        sw = (sw0, sw1)

        def fire_gather(b, s):
            pltpu.async_copy(table_hbm.at[hid_v.at[b]], h_v.at[s], sg[s])
            pltpu.async_copy(table_hbm.at[tid_v.at[b]], t_v.at[s], sg[s])

        def wait_gather(s):
            pltpu.make_async_copy(
                table_hbm.at[hid_v.at[0]], h_v.at[s], sg[s]).wait()
            pltpu.make_async_copy(
                table_hbm.at[tid_v.at[0]], t_v.at[s], sg[s]).wait()

        def wait_write(s):
            pltpu.make_async_copy(
                q_v.at[s], q_hbm.at[pl.ds(0, BLK)], sw[s]).wait()

        fire_gather(0, 0)

        def mul_block(s):
            def mul_row(r, c2):
                for j in range(D_CONST // (2 * LANES)):
                    s0 = pl.ds((2 * j) * LANES, LANES)
                    s1 = pl.ds((2 * j + 1) * LANES, LANES)
                    u0 = plsc.bitcast(h_v[s, r, s0] * t_v[s, r, s0],
                                      jnp.uint32)
                    u1 = plsc.bitcast(h_v[s, r, s1] * t_v[s, r, s1],
                                      jnp.uint32)
                    word = (u1 & jnp.uint32(0xFFFF0000)) | (u0 >> 16)
                    q_v[s, r, pl.ds(j * LANES, LANES)] = plsc.bitcast(
                        word, jnp.int32)
                return c2
            lax.fori_loop(0, BLK, mul_row, 0)

        def step(b, s):
            @pl.when(b + 1 <= NBLK - 1)
            def _():
                fire_gather(b + 1, 1 - s)

            wait_gather(s)

            @pl.when(b >= 2)
            def _():
                wait_write(s)

            mul_block(s)
            off = pl.multiple_of(base + b * BLK, BLK)
            pltpu.async_copy(q_v.at[s], q_hbm.at[pl.ds(off, BLK)], sw[s])

        def pair_body(i, carry):
            for s in range(2):
                step(2 * i + s, s)
            return carry

        lax.fori_loop(0, NBLK // 2, pair_body, 0)
        wait_write(0)
        wait_write(1)

    return k(x_default, hpos3, tpos3, constant_table)


BT = 16384


def _tc_project_chunk(c, pred_c, q_c, predicate_rel, w_proj, b2, out_prev):
    pred3 = pred_c.reshape(CA // BT, 1, BT)

    def body(pred_ref, q_ref, prel_ref, w_ref, b_ref, *rest):
        out_ref = rest[-1]
        pred = pred_ref[0, 0, :]
        oh = (pred[:, None]
              == lax.broadcasted_iota(jnp.int32, (BT, N_PREDS), 1)
              ).astype(jnp.float32)
        r = jnp.dot(oh, prel_ref[...], preferred_element_type=jnp.float32)
        qw = q_ref[...]
        f_lo = lax.bitcast_convert_type(qw << 16, jnp.float32)
        f_hi = lax.bitcast_convert_type(
            qw & jnp.int32(-65536), jnp.float32)
        q = jnp.concatenate([f_lo, f_hi], axis=1)
        inter = q * r
        acc = lax.dot_general(w_ref[...], inter, (((0,), (1,)), ((), ())),
                              preferred_element_type=jnp.float32)
        out_ref[...] = jnp.tanh(acc + b_ref[...])

    in_specs = [
        pl.BlockSpec((1, 1, BT), lambda i: (i, 0, 0)),
        pl.BlockSpec((BT, D_CONST // 2), lambda i: (i, 0)),
        pl.BlockSpec((N_PREDS, D_CONST), lambda i: (0, 0)),
        pl.BlockSpec((D_CONST, D_ATOM), lambda i: (0, 0)),
        pl.BlockSpec((D_ATOM, 1), lambda i: (0, 0)),
    ]
    args = [pred3, q_c, predicate_rel, w_proj, b2]
    aliases = {}
    if out_prev is not None:
        in_specs.append(pl.BlockSpec(memory_space=pl.ANY))
        args.append(out_prev)
        aliases = {5: 0}
    return pl.pallas_call(
        body,
        grid=(CA // BT,),
        in_specs=in_specs,
        out_specs=pl.BlockSpec((D_ATOM, BT),
                               lambda i, c=c: (0, c * (CA // BT) + i)),
        out_shape=jax.ShapeDtypeStruct((D_ATOM, N_ATOMS), jnp.float32),
        input_output_aliases=aliases,
    )(*args)


def _pack_perm():
    k = jnp.arange(D_CONST)
    half = k // (D_CONST // 2)
    k2 = k % (D_CONST // 2)
    return 32 * (k2 // 16) + (k2 % 16) + 16 * half


def kernel(X_default, pred_ids, head_pos, tail_pos, constant_table,
           predicate_rel, W_proj, b_proj):
    X_default = X_default.astype(jnp.int32)
    pred_ids = pred_ids.astype(jnp.int32)
    head_pos = head_pos.astype(jnp.int32)
    tail_pos = tail_pos.astype(jnp.int32)
    perm = _pack_perm()
    prel_p = predicate_rel[:, perm]
    w_p = W_proj[perm, :]
    b2 = b_proj.reshape(D_ATOM, 1)
    qs = [
        _sc_interact(X_default,
                     lax.slice_in_dim(head_pos, c * CA, (c + 1) * CA),
                     lax.slice_in_dim(tail_pos, c * CA, (c + 1) * CA),
                     constant_table)
        for c in range(N_CHUNKS)
    ]
    out_t = None
    for c in range(N_CHUNKS):
        pred_c = lax.slice_in_dim(pred_ids, c * CA, (c + 1) * CA)
        out_t = _tc_project_chunk(c, pred_c, qs[c], prel_p, w_p, b2, out_t)
    return out_t.T


# speedup vs baseline: 1.0207x; 1.0207x over previous
import functools

import jax
import jax.numpy as jnp
from jax import lax
from jax.experimental import pallas as pl
from jax.experimental.pallas import tpu as pltpu
from jax.experimental.pallas import tpu_sc as plsc

VOCAB = 100000
N_CONSTS = 16384
N_ATOMS = 65536
N_PREDS = 64
D_CONST = 128
D_ATOM = 64

NC = 2
NS = 16
NW = NC * NS
LANES = 16
BLK = 128
CHUNK_SIZES = (49152, 16384)


def _sc_interact(x_default, head_pos, tail_pos, constant_table):
    ca = head_pos.shape[0]
    CHUNK = ca // NW
    NBLK = CHUNK // BLK
    mesh = plsc.VectorSubcoreMesh(core_axis_name="c", subcore_axis_name="s",
                                  num_cores=2)
    hpos3 = head_pos.reshape(NW, NBLK, BLK)
    tpos3 = tail_pos.reshape(NW, NBLK, BLK)

    @functools.partial(
        pl.kernel,
        mesh=mesh,
        compiler_params=pltpu.CompilerParams(needs_layout_passes=False),
        out_type=jax.ShapeDtypeStruct((ca, D_CONST // 2), jnp.int32),
        scratch_types=[
            pltpu.VMEM((NBLK, BLK), jnp.int32),
            pltpu.VMEM((NBLK, BLK), jnp.int32),
            pltpu.VMEM((NBLK, BLK), jnp.int32),
            pltpu.VMEM((NBLK, BLK), jnp.int32),
            pltpu.VMEM((2, BLK, D_CONST), jnp.float32),
            pltpu.VMEM((2, BLK, D_CONST), jnp.float32),
            pltpu.VMEM((2, BLK, D_CONST // 2), jnp.int32),
            pltpu.SemaphoreType.DMA,
            pltpu.SemaphoreType.DMA,
            pltpu.SemaphoreType.DMA,
            pltpu.SemaphoreType.DMA,
            pltpu.SemaphoreType.DMA,
        ],
    )
    def k(xdef_hbm, hpos_hbm, tpos_hbm, table_hbm, q_hbm,
          hpos_v, tpos_v, hid_v, tid_v, h_v, t_v, q_v,
          sg0, sg1, sw0, sw1, sem_t):
        wid = lax.axis_index("s") * NC + lax.axis_index("c")
        base = pl.multiple_of(wid * CHUNK, CHUNK)
        pltpu.sync_copy(hpos_hbm.at[wid], hpos_v)
        pltpu.sync_copy(tpos_hbm.at[wid], tpos_v)

        trans = []
        for j in range(NBLK):
            trans.append(
                pltpu.async_copy(xdef_hbm.at[hpos_v.at[j]], hid_v.at[j], sem_t))
            trans.append(
                pltpu.async_copy(xdef_hbm.at[tpos_v.at[j]], tid_v.at[j], sem_t))
        for cp in trans:
            cp.wait()

        sg = (sg0, sg1)
        sw = (sw0, sw1)

        def fire_gather(b, s):
            pltpu.async_copy(table_hbm.at[hid_v.at[b]], h_v.at[s], sg[s])
            pltpu.async_copy(table_hbm.at[tid_v.at[b]], t_v.at[s], sg[s])

        def wait_gather(s):
            pltpu.make_async_copy(
                table_hbm.at[hid_v.at[0]], h_v.at[s], sg[s]).wait()
            pltpu.make_async_copy(
                table_hbm.at[tid_v.at[0]], t_v.at[s], sg[s]).wait()

        def wait_write(s):
            pltpu.make_async_copy(
                q_v.at[s], q_hbm.at[pl.ds(0, BLK)], sw[s]).wait()

        fire_gather(0, 0)

        def mul_block(s):
            def mul_row(r, c2):
                for j in range(D_CONST // (2 * LANES)):
                    s0 = pl.ds((2 * j) * LANES, LANES)
                    s1 = pl.ds((2 * j + 1) * LANES, LANES)
                    u0 = plsc.bitcast(h_v[s, r, s0] * t_v[s, r, s0],
                                      jnp.uint32)
                    u1 = plsc.bitcast(h_v[s, r, s1] * t_v[s, r, s1],
                                      jnp.uint32)
                    word = (u1 & jnp.uint32(0xFFFF0000)) | (u0 >> 16)
                    q_v[s, r, pl.ds(j * LANES, LANES)] = plsc.bitcast(
                        word, jnp.int32)
                return c2
            lax.fori_loop(0, BLK, mul_row, 0)

        def step(b, s):
            @pl.when(b + 1 <= NBLK - 1)
            def _():
                fire_gather(b + 1, 1 - s)

            wait_gather(s)

            @pl.when(b >= 2)
            def _():
                wait_write(s)

            mul_block(s)
            off = pl.multiple_of(base + b * BLK, BLK)
            pltpu.async_copy(q_v.at[s], q_hbm.at[pl.ds(off, BLK)], sw[s])

        def pair_body(i, carry):
            for s in range(2):
                step(2 * i + s, s)
            return carry

        lax.fori_loop(0, NBLK // 2, pair_body, 0)
        wait_write(0)
        wait_write(1)

    return k(x_default, hpos3, tpos3, constant_table)


BT = 16384


def _tc_project_chunk(start, pred_c, q_c, predicate_rel, w_proj, b2, out_prev):
    n = pred_c.shape[0]
    bt = min(BT, n)
    off_bt = start // bt
    pred3 = pred_c.reshape(n // bt, 1, bt)

    def body(pred_ref, q_ref, prel_ref, w_ref, b_ref, *rest):
        out_ref = rest[-1]
        pred = pred_ref[0, 0, :]
        oh = (pred[:, None]
              == lax.broadcasted_iota(jnp.int32, (bt, N_PREDS), 1)
              ).astype(jnp.float32)
        r = jnp.dot(oh, prel_ref[...], preferred_element_type=jnp.float32)
        qw = q_ref[...]
        f_lo = lax.bitcast_convert_type(qw << 16, jnp.float32)
        f_hi = lax.bitcast_convert_type(
            qw & jnp.int32(-65536), jnp.float32)
        q = jnp.concatenate([f_lo, f_hi], axis=1)
        inter = q * r
        acc = lax.dot_general(w_ref[...], inter, (((0,), (1,)), ((), ())),
                              preferred_element_type=jnp.float32)
        out_ref[...] = jnp.tanh(acc + b_ref[...])

    in_specs = [
        pl.BlockSpec((1, 1, bt), lambda i: (i, 0, 0)),
        pl.BlockSpec((bt, D_CONST // 2), lambda i: (i, 0)),
        pl.BlockSpec((N_PREDS, D_CONST), lambda i: (0, 0)),
        pl.BlockSpec((D_CONST, D_ATOM), lambda i: (0, 0)),
        pl.BlockSpec((D_ATOM, 1), lambda i: (0, 0)),
    ]
    args = [pred3, q_c, predicate_rel, w_proj, b2]
    aliases = {}
    if out_prev is not None:
        in_specs.append(pl.BlockSpec(memory_space=pl.ANY))
        args.append(out_prev)
        aliases = {5: 0}
    return pl.pallas_call(
        body,
        grid=(n // bt,),
        in_specs=in_specs,
        out_specs=pl.BlockSpec((D_ATOM, bt),
                               lambda i: (0, off_bt + i)),
        out_shape=jax.ShapeDtypeStruct((D_ATOM, N_ATOMS), jnp.float32),
        input_output_aliases=aliases,
    )(*args)


def _pack_perm():
    k = jnp.arange(D_CONST)
    half = k // (D_CONST // 2)
    k2 = k % (D_CONST // 2)
    return 32 * (k2 // 16) + (k2 % 16) + 16 * half


def kernel(X_default, pred_ids, head_pos, tail_pos, constant_table,
           predicate_rel, W_proj, b_proj):
    X_default = X_default.astype(jnp.int32)
    pred_ids = pred_ids.astype(jnp.int32)
    head_pos = head_pos.astype(jnp.int32)
    tail_pos = tail_pos.astype(jnp.int32)
    perm = _pack_perm()
    prel_p = predicate_rel[:, perm]
    w_p = W_proj[perm, :]
    b2 = b_proj.reshape(D_ATOM, 1)
    starts = [sum(CHUNK_SIZES[:i]) for i in range(len(CHUNK_SIZES))]
    qs = [
        _sc_interact(X_default,
                     lax.slice_in_dim(head_pos, st, st + n),
                     lax.slice_in_dim(tail_pos, st, st + n),
                     constant_table)
        for st, n in zip(starts, CHUNK_SIZES)
    ]
    out_t = None
    for qc, st, n in zip(qs, starts, CHUNK_SIZES):
        pred_c = lax.slice_in_dim(pred_ids, st, st + n)
        out_t = _tc_project_chunk(st, pred_c, qc, prel_p, w_p, b2, out_t)
    return out_t.T


# speedup vs baseline: 1.0682x; 1.0466x over previous
import functools

import jax
import jax.numpy as jnp
from jax import lax
from jax.experimental import pallas as pl
from jax.experimental.pallas import tpu as pltpu
from jax.experimental.pallas import tpu_sc as plsc

VOCAB = 100000
N_CONSTS = 16384
N_ATOMS = 65536
N_PREDS = 64
D_CONST = 128
D_ATOM = 64

NC = 2
NS = 16
NW = NC * NS
LANES = 16
CHUNK = N_ATOMS // NW
BLK = 128
NBLK = CHUNK // BLK


def _sc_interact(x_default, head_pos, tail_pos, constant_table):
    mesh = plsc.VectorSubcoreMesh(core_axis_name="c", subcore_axis_name="s",
                                  num_cores=2)
    hpos3 = head_pos.reshape(NW, NBLK, BLK)
    tpos3 = tail_pos.reshape(NW, NBLK, BLK)

    @functools.partial(
        pl.kernel,
        mesh=mesh,
        out_type=jax.ShapeDtypeStruct((N_ATOMS, D_CONST), jnp.float32),
        scratch_types=[
            pltpu.VMEM((NBLK, BLK), jnp.int32),
            pltpu.VMEM((NBLK, BLK), jnp.int32),
            pltpu.VMEM((NBLK, BLK), jnp.int32),
            pltpu.VMEM((NBLK, BLK), jnp.int32),
            pltpu.VMEM((3, BLK, D_CONST), jnp.float32),
            pltpu.VMEM((3, BLK, D_CONST), jnp.float32),
            pltpu.SemaphoreType.DMA,
            pltpu.SemaphoreType.DMA,
            pltpu.SemaphoreType.DMA,
            pltpu.SemaphoreType.DMA,
            pltpu.SemaphoreType.DMA,
            pltpu.SemaphoreType.DMA,
            pltpu.SemaphoreType.DMA,
        ],
    )
    def k(xdef_hbm, hpos_hbm, tpos_hbm, table_hbm, q_hbm,
          hpos_v, tpos_v, hid_v, tid_v, h_v, t_v,
          sg0, sg1, sg2, sw0, sw1, sw2, sem_t):
        wid = lax.axis_index("s") * NC + lax.axis_index("c")
        base = pl.multiple_of(wid * CHUNK, CHUNK)
        pltpu.sync_copy(hpos_hbm.at[wid], hpos_v)
        pltpu.sync_copy(tpos_hbm.at[wid], tpos_v)

        trans = []
        for j in range(NBLK):
            trans.append(
                pltpu.async_copy(xdef_hbm.at[hpos_v.at[j]], hid_v.at[j], sem_t))
            trans.append(
                pltpu.async_copy(xdef_hbm.at[tpos_v.at[j]], tid_v.at[j], sem_t))
        for cp in trans:
            cp.wait()

        sg = (sg0, sg1, sg2)
        sw = (sw0, sw1, sw2)

        def fire_gather(b, s):
            pltpu.async_copy(table_hbm.at[hid_v.at[b]], h_v.at[s], sg[s])
            pltpu.async_copy(table_hbm.at[tid_v.at[b]], t_v.at[s], sg[s])

        def wait_gather(s):
            pltpu.make_async_copy(
                table_hbm.at[hid_v.at[0]], h_v.at[s], sg[s]).wait()
            pltpu.make_async_copy(
                table_hbm.at[tid_v.at[0]], t_v.at[s], sg[s]).wait()

        def wait_write(s):
            pltpu.make_async_copy(
                h_v.at[s], q_hbm.at[pl.ds(0, BLK)], sw[s]).wait()

        fire_gather(0, 0)
        fire_gather(1, 1)

        def mul_block(s):
            def mul_row(r, c2):
                for j in range(D_CONST // LANES):
                    sl = pl.ds(j * LANES, LANES)
                    h_v[s, r, sl] = h_v[s, r, sl] * t_v[s, r, sl]
                return c2
            lax.fori_loop(0, BLK, mul_row, 0)

        def step(b, s):
            wait_gather(s)
            mul_block(s)

            @pl.when(b >= 1)
            def _():
                wait_write((s - 1) % 3)

            @pl.when(b + 2 <= NBLK - 1)
            def _():
                fire_gather(b + 2, (s + 2) % 3)

            off = pl.multiple_of(base + b * BLK, BLK)
            pltpu.async_copy(h_v.at[s], q_hbm.at[pl.ds(off, BLK)], sw[s])

        def tri_body(i, carry):
            for s in range(3):
                step(3 * i + s, s)
            return carry

        lax.fori_loop(0, NBLK // 3, tri_body, 0)
        step(NBLK - 1, (NBLK - 1) % 3)
        wait_write((NBLK - 1) % 3)

    return k(x_default, hpos3, tpos3, constant_table)


BT = 16384


def _tc_project(pred_ids, q, predicate_rel, w_proj, b_proj):
    pred3 = pred_ids.reshape(N_ATOMS // BT, 1, BT)
    b2 = b_proj.reshape(D_ATOM, 1)

    def body(pred_ref, q_ref, prel_ref, w_ref, b_ref, out_ref):
        pred = pred_ref[0, 0, :]
        oh = (pred[:, None]
              == lax.broadcasted_iota(jnp.int32, (BT, N_PREDS), 1)
              ).astype(jnp.float32)
        r = jnp.dot(oh, prel_ref[...], preferred_element_type=jnp.float32)
        inter = q_ref[...] * r
        acc = lax.dot_general(w_ref[...], inter, (((0,), (1,)), ((), ())),
                              preferred_element_type=jnp.float32)
        out_ref[...] = jnp.tanh(acc + b_ref[...])

    out_t = pl.pallas_call(
        body,
        grid=(N_ATOMS // BT,),
        in_specs=[
            pl.BlockSpec((1, 1, BT), lambda i: (i, 0, 0)),
            pl.BlockSpec((BT, D_CONST), lambda i: (i, 0)),
            pl.BlockSpec((N_PREDS, D_CONST), lambda i: (0, 0)),
            pl.BlockSpec((D_CONST, D_ATOM), lambda i: (0, 0)),
            pl.BlockSpec((D_ATOM, 1), lambda i: (0, 0)),
        ],
        out_specs=pl.BlockSpec((D_ATOM, BT), lambda i: (0, i)),
        out_shape=jax.ShapeDtypeStruct((D_ATOM, N_ATOMS), jnp.float32),
    )(pred3, q, predicate_rel, w_proj, b2)
    return out_t.T


def kernel(X_default, pred_ids, head_pos, tail_pos, constant_table,
           predicate_rel, W_proj, b_proj):
    X_default = X_default.astype(jnp.int32)
    pred_ids = pred_ids.astype(jnp.int32)
    head_pos = head_pos.astype(jnp.int32)
    tail_pos = tail_pos.astype(jnp.int32)
    q = _sc_interact(X_default, head_pos, tail_pos, constant_table)
    return _tc_project(pred_ids, q, predicate_rel, W_proj, b_proj)
